# baseline (speedup 1.0000x reference)
import functools

import jax
import jax.numpy as jnp
from jax import lax
from jax.experimental import pallas as pl
from jax.experimental.pallas import tpu as pltpu
from jax.experimental.pallas import tpu_sc as plsc

_BINS = 255
_LOW = -20.0
_HIGH = 20.0
_STEP = (_HIGH - _LOW) / (_BINS - 1)
_BLK = 8192

_NW = 32
_CHUNK = 128


def _prep_body(a_ref, idxb_ref, idxa_ref, wb_ref, wa_ref):
    a = a_ref[...]
    pb = a.shape[1]
    t = jnp.sign(a) * jnp.log(jnp.abs(a) + 1.0)
    u = (t - _LOW) * (1.0 / _STEP)
    u = jnp.clip(u, 0.0, float(_BINS - 1))
    kb = jnp.minimum(jnp.floor(u), float(_BINS - 2))
    wa = u - kb
    wb = 1.0 - wa
    kbi = kb.astype(jnp.int32)
    row0 = pl.program_id(0) * pb
    rows = row0 + jax.lax.broadcasted_iota(jnp.int32, (1, pb), 1)
    base = rows * _BINS + kbi
    idxb_ref[...] = base
    idxa_ref[...] = base + 1
    wb_ref[...] = wb
    wa_ref[...] = wa


def _gather_stage(table_flat, idxb, idxa, wb, wa, n):
    b_per_w = n // _NW
    k = b_per_w // _CHUNK
    mesh = plsc.VectorSubcoreMesh(core_axis_name="c", subcore_axis_name="s")

    @functools.partial(
        pl.kernel,
        mesh=mesh,
        out_type=jax.ShapeDtypeStruct((_NW, k, _CHUNK), jnp.float32),
        scratch_types=[
            pltpu.VMEM((k, _CHUNK), jnp.int32),
            pltpu.VMEM((k, _CHUNK), jnp.int32),
            pltpu.VMEM((k, _CHUNK), jnp.float32),
            pltpu.VMEM((k, _CHUNK), jnp.float32),
            pltpu.VMEM((_CHUNK,), jnp.float32),
            pltpu.VMEM((_CHUNK,), jnp.float32),
            pltpu.VMEM((k, _CHUNK), jnp.float32),
            pltpu.SemaphoreType.DMA,
            pltpu.SemaphoreType.DMA,
        ],
    )
    def g(table_hbm, idxb_hbm, idxa_hbm, wb_hbm, wa_hbm, out_hbm,
          idxb_v, idxa_v, wb_v, wa_v, gb_v, ga_v, scv_v, s1, s2):
        wid = lax.axis_index("s") * 2 + lax.axis_index("c")
        pltpu.sync_copy(idxb_hbm.at[wid], idxb_v)
        pltpu.sync_copy(idxa_hbm.at[wid], idxa_v)
        pltpu.sync_copy(wb_hbm.at[wid], wb_v)
        pltpu.sync_copy(wa_hbm.at[wid], wa_v)

        def body(j, carry):
            cb = pltpu.async_copy(table_hbm.at[idxb_v.at[j]], gb_v, s1)
            ca = pltpu.async_copy(table_hbm.at[idxa_v.at[j]], ga_v, s2)
            cb.wait()
            ca.wait()
            for c in range(_CHUNK // 16):
                sl = pl.ds(c * 16, 16)
                scv_v[j, sl] = wb_v[j, sl] * gb_v[sl] + wa_v[j, sl] * ga_v[sl]
            return carry

        lax.fori_loop(0, k, body, 0)
        pltpu.sync_copy(scv_v, out_hbm.at[wid])

    return g(table_flat, idxb.reshape(_NW, k, _CHUNK),
             idxa.reshape(_NW, k, _CHUNK), wb.reshape(_NW, k, _CHUNK),
             wa.reshape(_NW, k, _CHUNK))


def _main_body(logits_ref, scval_ref, out_ref):
    x = logits_ref[...]
    sc = scval_ref[...]
    e = jnp.exp(x)
    ones = jnp.ones((_BINS, 1), dtype=jnp.float32)
    s = jax.lax.dot(e, ones)
    lse = jnp.log(s)
    out_ref[...] = sc - jnp.swapaxes(lse, 0, 1)


def kernel(logits, actions, bins):
    del bins
    n = logits.shape[0]
    a_row = actions.reshape(1, n)

    idxb, idxa, wb, wa = pl.pallas_call(
        _prep_body,
        grid=(1,),
        in_specs=[pl.BlockSpec((1, n), lambda i: (0, 0))],
        out_specs=[pl.BlockSpec((1, n), lambda i: (0, 0))] * 4,
        out_shape=[
            jax.ShapeDtypeStruct((1, n), jnp.int32),
            jax.ShapeDtypeStruct((1, n), jnp.int32),
            jax.ShapeDtypeStruct((1, n), jnp.float32),
            jax.ShapeDtypeStruct((1, n), jnp.float32),
        ],
    )(a_row)

    scval = _gather_stage(logits.reshape(n * _BINS), idxb, idxa, wb, wa, n)
    scval_row = scval.reshape(1, n)

    out = pl.pallas_call(
        _main_body,
        grid=(n // _BLK,),
        in_specs=[
            pl.BlockSpec((_BLK, _BINS), lambda i: (i, 0)),
            pl.BlockSpec((1, _BLK), lambda i: (0, i)),
        ],
        out_specs=pl.BlockSpec((1, _BLK), lambda i: (0, i)),
        out_shape=jax.ShapeDtypeStruct((1, n), logits.dtype),
        compiler_params=pltpu.CompilerParams(
            dimension_semantics=("arbitrary",),
        ),
    )(logits, scval_row)
    return out.reshape(n, 1)


# speedup vs baseline: 1.1358x; 1.1358x over previous
import functools

import jax
import jax.numpy as jnp
from jax import lax
from jax.experimental import pallas as pl
from jax.experimental.pallas import tpu as pltpu
from jax.experimental.pallas import tpu_sc as plsc

_BINS = 255
_LOW = -20.0
_HIGH = 20.0
_STEP = (_HIGH - _LOW) / (_BINS - 1)
_BLK = 8192

_NW = 32
_CHUNK = 128


def _prep_body(a_ref, idxb_ref, idxa_ref, wb_ref, wa_ref):
    a = a_ref[...]
    pb = a.shape[1]
    t = jnp.sign(a) * jnp.log(jnp.abs(a) + 1.0)
    u = (t - _LOW) * (1.0 / _STEP)
    u = jnp.clip(u, 0.0, float(_BINS - 1))
    kb = jnp.minimum(jnp.floor(u), float(_BINS - 2))
    wa = u - kb
    wb = 1.0 - wa
    kbi = kb.astype(jnp.int32)
    row0 = pl.program_id(0) * pb
    rows = row0 + jax.lax.broadcasted_iota(jnp.int32, (1, pb), 1)
    base = rows * _BINS + kbi
    idxb_ref[...] = base
    idxa_ref[...] = base + 1
    wb_ref[...] = wb
    wa_ref[...] = wa


def _gather_stage(table_flat, idxb, idxa, wb, wa, n):
    b_per_w = n // _NW
    k = b_per_w // _CHUNK
    mesh = plsc.VectorSubcoreMesh(core_axis_name="c", subcore_axis_name="s")

    @functools.partial(
        pl.kernel,
        mesh=mesh,
        out_type=jax.ShapeDtypeStruct((_NW, k, _CHUNK), jnp.float32),
        scratch_types=[
            pltpu.VMEM((k, _CHUNK), jnp.int32),
            pltpu.VMEM((k, _CHUNK), jnp.int32),
            pltpu.VMEM((k, _CHUNK), jnp.float32),
            pltpu.VMEM((k, _CHUNK), jnp.float32),
            pltpu.VMEM((_CHUNK,), jnp.float32),
            pltpu.VMEM((_CHUNK,), jnp.float32),
            pltpu.VMEM((k, _CHUNK), jnp.float32),
            pltpu.SemaphoreType.DMA,
            pltpu.SemaphoreType.DMA,
        ],
    )
    def g(table_hbm, idxb_hbm, idxa_hbm, wb_hbm, wa_hbm, out_hbm,
          idxb_v, idxa_v, wb_v, wa_v, gb_v, ga_v, scv_v, s1, s2):
        wid = lax.axis_index("s") * 2 + lax.axis_index("c")
        pltpu.sync_copy(idxb_hbm.at[wid], idxb_v)
        pltpu.sync_copy(idxa_hbm.at[wid], idxa_v)
        pltpu.sync_copy(wb_hbm.at[wid], wb_v)
        pltpu.sync_copy(wa_hbm.at[wid], wa_v)

        def body(j, carry):
            cb = pltpu.async_copy(table_hbm.at[idxb_v.at[j]], gb_v, s1)
            ca = pltpu.async_copy(table_hbm.at[idxa_v.at[j]], ga_v, s2)
            cb.wait()
            ca.wait()
            for c in range(_CHUNK // 16):
                sl = pl.ds(c * 16, 16)
                scv_v[j, sl] = wb_v[j, sl] * gb_v[sl] + wa_v[j, sl] * ga_v[sl]
            return carry

        lax.fori_loop(0, k, body, 0)
        pltpu.sync_copy(scv_v, out_hbm.at[wid])

    return g(table_flat, idxb.reshape(_NW, k, _CHUNK),
             idxa.reshape(_NW, k, _CHUNK), wb.reshape(_NW, k, _CHUNK),
             wa.reshape(_NW, k, _CHUNK))


def _lse_body(logits_ref, out_ref):
    x = logits_ref[...]
    e = jnp.exp(x)
    ones = jnp.ones((_BINS, 1), dtype=jnp.float32)
    s = jax.lax.dot(e, ones)
    lse = jnp.log(s)
    out_ref[...] = jnp.swapaxes(lse, 0, 1)


def _combine_body(sc_ref, lse_ref, out_ref):
    out_ref[...] = sc_ref[...] - lse_ref[...]


def kernel(logits, actions, bins):
    del bins
    n = logits.shape[0]
    a_row = actions.reshape(1, n)

    idxb, idxa, wb, wa = pl.pallas_call(
        _prep_body,
        grid=(1,),
        in_specs=[pl.BlockSpec((1, n), lambda i: (0, 0))],
        out_specs=[pl.BlockSpec((1, n), lambda i: (0, 0))] * 4,
        out_shape=[
            jax.ShapeDtypeStruct((1, n), jnp.int32),
            jax.ShapeDtypeStruct((1, n), jnp.int32),
            jax.ShapeDtypeStruct((1, n), jnp.float32),
            jax.ShapeDtypeStruct((1, n), jnp.float32),
        ],
    )(a_row)

    scval = _gather_stage(logits.reshape(n * _BINS), idxb, idxa, wb, wa, n)
    scval_row = scval.reshape(1, n)

    lse_row = pl.pallas_call(
        _lse_body,
        grid=(n // _BLK,),
        in_specs=[pl.BlockSpec((_BLK, _BINS), lambda i: (i, 0))],
        out_specs=pl.BlockSpec((1, _BLK), lambda i: (0, i)),
        out_shape=jax.ShapeDtypeStruct((1, n), logits.dtype),
        compiler_params=pltpu.CompilerParams(
            dimension_semantics=("arbitrary",),
        ),
    )(logits)

    out = pl.pallas_call(
        _combine_body,
        grid=(1,),
        in_specs=[
            pl.BlockSpec((1, n), lambda i: (0, 0)),
            pl.BlockSpec((1, n), lambda i: (0, 0)),
        ],
        out_specs=pl.BlockSpec((1, n), lambda i: (0, 0)),
        out_shape=jax.ShapeDtypeStruct((1, n), logits.dtype),
    )(scval_row, lse_row)
    return out.reshape(n, 1)
